# baseline (speedup 1.0000x reference)
import functools

import jax
import jax.numpy as jnp
from jax import lax
from jax.experimental import pallas as pl
from jax.experimental.pallas import tpu as pltpu
from jax.experimental.pallas import tpu_sc as plsc

N = 50000
CIN = 128
COUT = 128
D = 64
DP = D + 2
KVOL = 27

NC = 2
NS = 16
NW = NC * NS
R = 1664
NP = NW * R
R2 = NP // NS
RC = 104
NCH = R // RC
NBUF = 6

TBL = DP * DP * DP
TPT = 18176
TSZ = NS * TPT
ZROW = N
HOFF = DP * DP + DP + 1

_mesh = plsc.VectorSubcoreMesh(core_axis_name="c", subcore_axis_name="s")


def _iota16():
    return lax.iota(jnp.int32, 16)


def _k1_body(cx_h, cy_h, cz_h, ztbl_h, tbl_h,
             cxv, cyv, czv, idv, hsv, idv2, hsv2, zbuf,
             sem_a, sem_b, sem_ld):
    c = lax.axis_index("c")
    s = lax.axis_index("s")
    cbase = c * TSZ

    pltpu.async_copy(ztbl_h.at[pl.ds(s * TPT, TPT)], zbuf, sem_a).wait()
    pltpu.async_copy(
        zbuf, tbl_h.at[pl.ds(cbase + s * TPT, TPT)], sem_a).wait()
    plsc.subcore_barrier()

    sbase = s * R2
    d1 = pltpu.async_copy(cx_h.at[pl.ds(sbase, R2)], cxv, sem_ld)
    d2 = pltpu.async_copy(cy_h.at[pl.ds(sbase, R2)], cyv, sem_ld)
    d3 = pltpu.async_copy(cz_h.at[pl.ds(sbase, R2)], czv, sem_ld)
    d1.wait()
    d2.wait()
    d3.wait()

    def sc_grp(g, _):
        off = g * 16
        p16 = sbase + off + _iota16()
        h16 = (cxv[pl.ds(off, 16)] * (DP * DP) + cyv[pl.ds(off, 16)] * DP
               + czv[pl.ds(off, 16)] + HOFF)
        h16 = jnp.where(p16 < N, h16, TBL + (p16 - N))
        hsv[pl.ds(off, 16)] = cbase + h16
        idv[pl.ds(off, 16)] = p16
        return 0

    lax.fori_loop(0, R // 16, sc_grp, 0)

    def sc_grp2(g, _):
        off = g * 16
        p16 = sbase + R + off + _iota16()
        h16 = (cxv[pl.ds(R + off, 16)] * (DP * DP)
               + cyv[pl.ds(R + off, 16)] * DP
               + czv[pl.ds(R + off, 16)] + HOFF)
        h16 = jnp.where(p16 < N, h16, TBL + (p16 - N))
        hsv2[pl.ds(off, 16)] = cbase + h16
        idv2[pl.ds(off, 16)] = p16
        return 0

    lax.fori_loop(0, R // 16, sc_grp2, 0)
    db1 = pltpu.async_copy(idv, tbl_h.at[hsv], sem_b)
    db2 = pltpu.async_copy(idv2, tbl_h.at[hsv2], sem_b)
    db1.wait()
    db2.wait()


@functools.partial(
    pl.kernel,
    out_type=jax.ShapeDtypeStruct((NC * TSZ,), jnp.int32),
    mesh=_mesh,
    scratch_types=[
        pltpu.VMEM((R2,), jnp.int32),
        pltpu.VMEM((R2,), jnp.int32),
        pltpu.VMEM((R2,), jnp.int32),
        pltpu.VMEM((R,), jnp.int32),
        pltpu.VMEM((R,), jnp.int32),
        pltpu.VMEM((R,), jnp.int32),
        pltpu.VMEM((R,), jnp.int32),
        pltpu.VMEM((TPT,), jnp.int32),
        pltpu.SemaphoreType.DMA,
        pltpu.SemaphoreType.DMA,
        pltpu.SemaphoreType.DMA,
    ],
)
def _k1(*args):
    _k1_body(*args)


def _k2_body(cx_h, cy_h, cz_h, tbl_h, feats_h, g_h,
             cxv, cyv, czv, hv, nhvs, tvs, rbufs,
             sem_ld, sems_t, sems_g, sems_w):
    c = lax.axis_index("c")
    s = lax.axis_index("s")
    wid = s * NC + c
    cbase = c * TSZ
    base = wid * R
    d1 = pltpu.async_copy(cx_h.at[pl.ds(base, R)], cxv, sem_ld)
    d2 = pltpu.async_copy(cy_h.at[pl.ds(base, R)], cyv, sem_ld)
    d3 = pltpu.async_copy(cz_h.at[pl.ds(base, R)], czv, sem_ld)
    d1.wait()
    d2.wait()
    d3.wait()

    def h_grp(g, _):
        off = g * 16
        hv[pl.ds(off, 16)] = (cbase + HOFF
                              + cxv[pl.ds(off, 16)] * (DP * DP)
                              + cyv[pl.ds(off, 16)] * DP
                              + czv[pl.ds(off, 16)])
        return 0

    lax.fori_loop(0, R // 16, h_grp, 0)

    tg = [None] * KVOL

    K = NBUF - 1

    def rows_phase(o, tv):
        gd = [None] * NCH
        wd = [None] * NCH

        def fire_write(j):
            wd[j] = pltpu.async_copy(
                rbufs[j % NBUF],
                g_h.at[o, pl.ds(base + j * RC, RC)],
                sems_w[j % NBUF])

        for j in range(NCH):
            if j >= NBUF:
                wd[j - NBUF].wait()
            gd[j] = pltpu.async_copy(
                feats_h.at[tv.at[pl.ds(j * RC, RC)]], rbufs[j % NBUF],
                sems_g[j % NBUF])
            if j >= K:
                gd[j - K].wait()
                fire_write(j - K)
        for j in range(NCH - K, NCH):
            gd[j].wait()
            fire_write(j)
        for j in range(NCH - NBUF, NCH):
            wd[j].wait()

    for o in range(KVOL):
        dx = o // 9 - 1
        dy = (o // 3) % 3 - 1
        dz = o % 3 - 1
        c66 = dx * (DP * DP) + dy * DP + dz
        ob = o % 2
        nhv = nhvs[ob]

        def nh_grp(g, _):
            off = g * 16
            nhv[pl.ds(off, 16)] = hv[pl.ds(off, 16)] + c66
            return 0

        lax.fori_loop(0, R // 16, nh_grp, 0)
        tg[o] = pltpu.async_copy(tbl_h.at[nhv], tvs[ob], sems_t[ob])
        if o >= 1:
            tg[o - 1].wait()
            rows_phase(o - 1, tvs[(o - 1) % 2])
    tg[KVOL - 1].wait()
    rows_phase(KVOL - 1, tvs[(KVOL - 1) % 2])


@functools.partial(
    pl.kernel,
    out_type=jax.ShapeDtypeStruct((KVOL, NP, CIN), jnp.float32),
    mesh=_mesh,
    scratch_types=[
        pltpu.VMEM((R,), jnp.int32),
        pltpu.VMEM((R,), jnp.int32),
        pltpu.VMEM((R,), jnp.int32),
        pltpu.VMEM((R,), jnp.int32),
        [pltpu.VMEM((R,), jnp.int32)] * 2,
        [pltpu.VMEM((R,), jnp.int32)] * 2,
        [pltpu.VMEM((RC, CIN), jnp.float32)] * NBUF,
        pltpu.SemaphoreType.DMA,
        [pltpu.SemaphoreType.DMA] * 2,
        [pltpu.SemaphoreType.DMA] * NBUF,
        [pltpu.SemaphoreType.DMA] * NBUF,
    ],
)
def _k2(*args):
    _k2_body(*args)


BN = 512


def _k3_body(g_ref, w_ref, b_ref, out_ref):
    o = pl.program_id(1)

    @pl.when(o == 0)
    def _():
        out_ref[...] = jnp.broadcast_to(b_ref[0], (BN, COUT))

    out_ref[...] += jnp.dot(g_ref[0], w_ref[o],
                            preferred_element_type=jnp.float32)


def _k3(gb, Wb, b2):
    return pl.pallas_call(
        _k3_body,
        grid=(NP // BN, KVOL),
        in_specs=[
            pl.BlockSpec((1, BN, CIN), lambda i, o: (o, i, 0)),
            pl.BlockSpec((KVOL, CIN, COUT), lambda i, o: (0, 0, 0)),
            pl.BlockSpec((1, COUT), lambda i, o: (0, 0)),
        ],
        out_specs=pl.BlockSpec((BN, COUT), lambda i, o: (i, 0)),
        out_shape=jax.ShapeDtypeStruct((NP, COUT), jnp.float32),
        compiler_params=pltpu.CompilerParams(
            dimension_semantics=("arbitrary", "arbitrary")),
    )(gb, Wb, b2)


def kernel(feats, coords, W, b):
    pad = NP - N
    cx = jnp.pad(coords[:, 0], (0, pad))
    cy = jnp.pad(coords[:, 1], (0, pad))
    cz = jnp.pad(coords[:, 2], (0, pad))
    fpad = jnp.pad(feats, ((0, pad), (0, 0)))
    ztbl = jnp.full((TSZ,), ZROW, dtype=jnp.int32)
    tbl = _k1(cx, cy, cz, ztbl)
    g = _k2(cx, cy, cz, tbl, fpad)
    out = _k3(g, W, b.reshape(1, COUT))
    return out[:N]


# speedup vs baseline: 1.0030x; 1.0030x over previous
import functools

import jax
import jax.numpy as jnp
from jax import lax
from jax.experimental import pallas as pl
from jax.experimental.pallas import tpu as pltpu
from jax.experimental.pallas import tpu_sc as plsc

N = 50000
CIN = 128
COUT = 128
D = 64
DP = D + 2
KVOL = 27

NC = 2
NS = 16
NW = NC * NS
R = 1664
NP = NW * R
R2 = NP // NS
RC = 104
NCH = R // RC
NBUF = 6

TBL = DP * DP * DP
TPT = 18176
TSZ = NS * TPT
ZROW = N
HOFF = DP * DP + DP + 1

_mesh = plsc.VectorSubcoreMesh(core_axis_name="c", subcore_axis_name="s")


def _iota16():
    return lax.iota(jnp.int32, 16)


def _k1_body(cx_h, cy_h, cz_h, ztbl_h, tbl_h,
             cxv, cyv, czv, idv, hsv, idv2, hsv2, zbuf,
             sem_a, sem_b, sem_ld):
    c = lax.axis_index("c")
    s = lax.axis_index("s")
    cbase = c * TSZ

    pltpu.async_copy(ztbl_h.at[pl.ds(s * TPT, TPT)], zbuf, sem_a).wait()
    pltpu.async_copy(
        zbuf, tbl_h.at[pl.ds(cbase + s * TPT, TPT)], sem_a).wait()
    plsc.subcore_barrier()

    sbase = s * R2
    d1 = pltpu.async_copy(cx_h.at[pl.ds(sbase, R2)], cxv, sem_ld)
    d2 = pltpu.async_copy(cy_h.at[pl.ds(sbase, R2)], cyv, sem_ld)
    d3 = pltpu.async_copy(cz_h.at[pl.ds(sbase, R2)], czv, sem_ld)
    d1.wait()
    d2.wait()
    d3.wait()

    def sc_grp(g, _):
        off = g * 16
        p16 = sbase + off + _iota16()
        h16 = (cxv[pl.ds(off, 16)] * (DP * DP) + cyv[pl.ds(off, 16)] * DP
               + czv[pl.ds(off, 16)] + HOFF)
        h16 = jnp.where(p16 < N, h16, TBL + (p16 - N))
        hsv[pl.ds(off, 16)] = cbase + h16
        idv[pl.ds(off, 16)] = p16
        return 0

    lax.fori_loop(0, R // 16, sc_grp, 0)

    def sc_grp2(g, _):
        off = g * 16
        p16 = sbase + R + off + _iota16()
        h16 = (cxv[pl.ds(R + off, 16)] * (DP * DP)
               + cyv[pl.ds(R + off, 16)] * DP
               + czv[pl.ds(R + off, 16)] + HOFF)
        h16 = jnp.where(p16 < N, h16, TBL + (p16 - N))
        hsv2[pl.ds(off, 16)] = cbase + h16
        idv2[pl.ds(off, 16)] = p16
        return 0

    lax.fori_loop(0, R // 16, sc_grp2, 0)
    db1 = pltpu.async_copy(idv, tbl_h.at[hsv], sem_b)
    db2 = pltpu.async_copy(idv2, tbl_h.at[hsv2], sem_b)
    db1.wait()
    db2.wait()


@functools.partial(
    pl.kernel,
    out_type=jax.ShapeDtypeStruct((NC * TSZ,), jnp.int32),
    mesh=_mesh,
    scratch_types=[
        pltpu.VMEM((R2,), jnp.int32),
        pltpu.VMEM((R2,), jnp.int32),
        pltpu.VMEM((R2,), jnp.int32),
        pltpu.VMEM((R,), jnp.int32),
        pltpu.VMEM((R,), jnp.int32),
        pltpu.VMEM((R,), jnp.int32),
        pltpu.VMEM((R,), jnp.int32),
        pltpu.VMEM((TPT,), jnp.int32),
        pltpu.SemaphoreType.DMA,
        pltpu.SemaphoreType.DMA,
        pltpu.SemaphoreType.DMA,
    ],
)
def _k1(*args):
    _k1_body(*args)


def _k2_body(cx_h, cy_h, cz_h, tbl_h, feats_h, g_h,
             cxv, cyv, czv, hv, nhvs, tvs, rbufs,
             sem_ld, sems_t, sems_g, sems_w):
    c = lax.axis_index("c")
    s = lax.axis_index("s")
    wid = s * NC + c
    cbase = c * TSZ
    base = wid * R
    d1 = pltpu.async_copy(cx_h.at[pl.ds(base, R)], cxv, sem_ld)
    d2 = pltpu.async_copy(cy_h.at[pl.ds(base, R)], cyv, sem_ld)
    d3 = pltpu.async_copy(cz_h.at[pl.ds(base, R)], czv, sem_ld)
    d1.wait()
    d2.wait()
    d3.wait()

    def h_grp(g, _):
        off = g * 16
        hv[pl.ds(off, 16)] = (cbase + HOFF
                              + cxv[pl.ds(off, 16)] * (DP * DP)
                              + cyv[pl.ds(off, 16)] * DP
                              + czv[pl.ds(off, 16)])
        return 0

    lax.fori_loop(0, R // 16, h_grp, 0)

    tg = [None] * KVOL

    K = NBUF - 1

    def rows_phase(o, tv):
        gd = [None] * NCH
        wd = [None] * NCH

        def fire_write(j):
            wd[j] = pltpu.async_copy(
                rbufs[j % NBUF],
                g_h.at[o, pl.ds(base + j * RC, RC)],
                sems_w[j % NBUF])

        for j in range(NCH):
            if j >= NBUF:
                wd[j - NBUF].wait()
            src = (feats_h.at[pl.ds(base + j * RC, RC)] if o == 13
                   else feats_h.at[tv.at[pl.ds(j * RC, RC)]])
            gd[j] = pltpu.async_copy(src, rbufs[j % NBUF],
                                     sems_g[j % NBUF])
            if j >= K:
                gd[j - K].wait()
                fire_write(j - K)
        for j in range(NCH - K, NCH):
            gd[j].wait()
            fire_write(j)
        for j in range(NCH - NBUF, NCH):
            wd[j].wait()

    for o in range(KVOL):
        dx = o // 9 - 1
        dy = (o // 3) % 3 - 1
        dz = o % 3 - 1
        c66 = dx * (DP * DP) + dy * DP + dz
        ob = o % 2
        nhv = nhvs[ob]

        def nh_grp(g, _):
            off = g * 16
            nhv[pl.ds(off, 16)] = hv[pl.ds(off, 16)] + c66
            return 0

        lax.fori_loop(0, R // 16, nh_grp, 0)
        tg[o] = pltpu.async_copy(tbl_h.at[nhv], tvs[ob], sems_t[ob])
        if o >= 1:
            tg[o - 1].wait()
            rows_phase(o - 1, tvs[(o - 1) % 2])
    tg[KVOL - 1].wait()
    rows_phase(KVOL - 1, tvs[(KVOL - 1) % 2])


@functools.partial(
    pl.kernel,
    out_type=jax.ShapeDtypeStruct((KVOL, NP, CIN), jnp.float32),
    mesh=_mesh,
    scratch_types=[
        pltpu.VMEM((R,), jnp.int32),
        pltpu.VMEM((R,), jnp.int32),
        pltpu.VMEM((R,), jnp.int32),
        pltpu.VMEM((R,), jnp.int32),
        [pltpu.VMEM((R,), jnp.int32)] * 2,
        [pltpu.VMEM((R,), jnp.int32)] * 2,
        [pltpu.VMEM((RC, CIN), jnp.float32)] * NBUF,
        pltpu.SemaphoreType.DMA,
        [pltpu.SemaphoreType.DMA] * 2,
        [pltpu.SemaphoreType.DMA] * NBUF,
        [pltpu.SemaphoreType.DMA] * NBUF,
    ],
)
def _k2(*args):
    _k2_body(*args)


BN = 512


def _k3_body(g_ref, w_ref, b_ref, out_ref):
    o = pl.program_id(1)

    @pl.when(o == 0)
    def _():
        out_ref[...] = jnp.broadcast_to(b_ref[0], (BN, COUT))

    out_ref[...] += jnp.dot(g_ref[0], w_ref[o],
                            preferred_element_type=jnp.float32)


def _k3(gb, Wb, b2):
    return pl.pallas_call(
        _k3_body,
        grid=(NP // BN, KVOL),
        in_specs=[
            pl.BlockSpec((1, BN, CIN), lambda i, o: (o, i, 0)),
            pl.BlockSpec((KVOL, CIN, COUT), lambda i, o: (0, 0, 0)),
            pl.BlockSpec((1, COUT), lambda i, o: (0, 0)),
        ],
        out_specs=pl.BlockSpec((BN, COUT), lambda i, o: (i, 0)),
        out_shape=jax.ShapeDtypeStruct((NP, COUT), jnp.float32),
        compiler_params=pltpu.CompilerParams(
            dimension_semantics=("arbitrary", "arbitrary")),
    )(gb, Wb, b2)


def kernel(feats, coords, W, b):
    pad = NP - N
    cx = jnp.pad(coords[:, 0], (0, pad))
    cy = jnp.pad(coords[:, 1], (0, pad))
    cz = jnp.pad(coords[:, 2], (0, pad))
    fpad = jnp.pad(feats, ((0, pad), (0, 0)))
    ztbl = jnp.full((TSZ,), ZROW, dtype=jnp.int32)
    tbl = _k1(cx, cy, cz, ztbl)
    g = _k2(cx, cy, cz, tbl, fpad)
    out = _k3(g, W, b.reshape(1, COUT))
    return out[:N]
